# Initial kernel scaffold; baseline (speedup 1.0000x reference)
import jax
import jax.numpy as jnp
from jax.experimental import pallas as pl


def kernel(x, edge_index, batch, ogt, W1, b1, W2, b2, Wo1, bo1, Wo2, bo2, Wf1, bf1, Wf2, bf2):
    raise NotImplementedError("write your pallas kernel here")


# speedup vs baseline: 15.6862x; 15.6862x over previous
import functools

import jax
import jax.numpy as jnp
from jax import lax
from jax.experimental import pallas as pl
from jax.experimental.pallas import tpu as pltpu
from jax.experimental.pallas import tpu_sc as plsc

N = 10000
E = 320000
D = 128
G = 16
NEG = 0.01

NC, NS = 2, 16
NW = NC * NS
K = 128
EPW = 10240
EPAD = NW * EPW
NCHUNK = EPW // K
NPAD = 10240
RPS = NPAD // NS
DEGW = 16

_HIGHEST = lax.Precision.HIGHEST

_mesh = plsc.VectorSubcoreMesh(
    core_axis_name="c", subcore_axis_name="s", num_cores=NC, num_subcores=NS
)


@functools.partial(
    pl.kernel,
    out_type=jax.ShapeDtypeStruct((NC * NPAD, DEGW), jnp.float32),
    mesh=_mesh,
    scratch_types=[
        pltpu.VMEM((K,), jnp.int32),
        pltpu.VMEM((K, DEGW), jnp.float32),
        pltpu.VMEM_SHARED((NPAD, DEGW), jnp.float32),
    ],
)
def _sc_degree(dst_hbm, ones_hbm, zeros_hbm, out_hbm, idx_v, ones_v, acc):
    cid = lax.axis_index("c")
    sid = lax.axis_index("s")
    wid = sid * NC + cid
    pltpu.sync_copy(zeros_hbm, acc.at[pl.ds(sid * RPS, RPS)])
    pltpu.sync_copy(ones_hbm, ones_v)
    plsc.subcore_barrier()
    base = wid * EPW

    @pl.loop(0, NCHUNK)
    def _(ci):
        pltpu.sync_copy(dst_hbm.at[pl.ds(base + ci * K, K)], idx_v)
        pltpu.sync_copy(ones_v, acc.at[idx_v], add=True)

    plsc.subcore_barrier()
    pltpu.sync_copy(
        acc.at[pl.ds(sid * RPS, RPS)],
        out_hbm.at[pl.ds(cid * NPAD + sid * RPS, RPS)],
    )


@functools.partial(
    pl.kernel,
    out_type=jax.ShapeDtypeStruct((NC * NPAD, D), jnp.float32),
    mesh=_mesh,
    scratch_types=[
        pltpu.VMEM((K,), jnp.int32),
        pltpu.VMEM((K,), jnp.int32),
        pltpu.VMEM((K, D), jnp.float32),
        pltpu.VMEM_SHARED((NPAD, D), jnp.float32),
    ],
)
def _sc_scatter(h_hbm, src_hbm, dst_hbm, zeros_hbm, out_hbm,
                src_v, dst_v, rows_v, acc):
    cid = lax.axis_index("c")
    sid = lax.axis_index("s")
    wid = sid * NC + cid
    pltpu.sync_copy(zeros_hbm, acc.at[pl.ds(sid * RPS, RPS)])
    plsc.subcore_barrier()
    base = wid * EPW

    @pl.loop(0, NCHUNK)
    def _(ci):
        off = base + ci * K
        pltpu.sync_copy(src_hbm.at[pl.ds(off, K)], src_v)
        pltpu.sync_copy(dst_hbm.at[pl.ds(off, K)], dst_v)
        pltpu.sync_copy(h_hbm.at[src_v], rows_v)
        pltpu.sync_copy(rows_v, acc.at[dst_v], add=True)

    plsc.subcore_barrier()
    pltpu.sync_copy(
        acc.at[pl.ds(sid * RPS, RPS)],
        out_hbm.at[pl.ds(cid * NPAD + sid * RPS, RPS)],
    )


def _leaky(v):
    return jnp.where(v >= 0, v, NEG * v)


def _dinv_from(degw_ref):
    deg = 1.0 + degw_ref[...][0:N, 0:1] + degw_ref[...][NPAD:NPAD + N, 0:1]
    return lax.rsqrt(deg)


def _mm_body(x_ref, w_ref, o_ref):
    o_ref[...] = jnp.dot(x_ref[...], w_ref[...],
                         preferred_element_type=jnp.float32,
                         precision=_HIGHEST)


_mm = pl.pallas_call(_mm_body, out_shape=jax.ShapeDtypeStruct((N, D), jnp.float32))


def _scale_body(xw_ref, degw_ref, o_ref):
    o_ref[...] = xw_ref[...] * _dinv_from(degw_ref)


_scale = pl.pallas_call(_scale_body,
                        out_shape=jax.ShapeDtypeStruct((N, D), jnp.float32))


def _mid_body(s_ref, hp_ref, degw_ref, b_ref, w_ref, o_ref):
    dinv = _dinv_from(degw_ref)
    s = s_ref[...][0:N, :] + s_ref[...][NPAD:NPAD + N, :]
    h = _leaky(dinv * (s + hp_ref[...]) + b_ref[...])
    o_ref[...] = jnp.dot(h, w_ref[...],
                         preferred_element_type=jnp.float32,
                         precision=_HIGHEST) * dinv


_mid = pl.pallas_call(_mid_body,
                      out_shape=jax.ShapeDtypeStruct((N, D), jnp.float32))


def _final_body(s_ref, hp_ref, degw_ref, b_ref, batch_ref, ogt_ref,
                wo1_ref, bo1_ref, wo2_ref, bo2_ref,
                wf1_ref, bf1_ref, wf2_ref, bf2_ref, o_ref):
    dinv = _dinv_from(degw_ref)
    s = s_ref[...][0:N, :] + s_ref[...][NPAD:NPAD + N, :]
    h = _leaky(dinv * (s + hp_ref[...]) + b_ref[...])
    gid = lax.broadcasted_iota(jnp.int32, (G, 1), 0)
    onehot = (batch_ref[...] == gid).astype(jnp.float32)
    sums = jnp.dot(onehot, h, preferred_element_type=jnp.float32,
                   precision=_HIGHEST)
    cnt = jnp.sum(onehot, axis=1, keepdims=True)
    pooled = sums / jnp.maximum(cnt, 1.0)
    o = ogt_ref[...]
    o = _leaky(jnp.dot(o, wo1_ref[...],
                       preferred_element_type=jnp.float32) + bo1_ref[...])
    o = _leaky(jnp.dot(o, wo2_ref[...],
                       preferred_element_type=jnp.float32) + bo2_ref[...])
    wf1 = wf1_ref[...]
    z = _leaky(jnp.dot(pooled, wf1[0:D, :],
                       preferred_element_type=jnp.float32)
               + jnp.dot(o, wf1[D:D + 10, :],
                         preferred_element_type=jnp.float32)
               + bf1_ref[...])
    o_ref[...] = jnp.dot(z, wf2_ref[...],
                         preferred_element_type=jnp.float32) + bf2_ref[...]


_final = pl.pallas_call(_final_body,
                        out_shape=jax.ShapeDtypeStruct((G, 1), jnp.float32))


def kernel(x, edge_index, batch, ogt, W1, b1, W2, b2,
           Wo1, bo1, Wo2, bo2, Wf1, bf1, Wf2, bf2):
    src = edge_index[0].astype(jnp.int32)
    dst = edge_index[1].astype(jnp.int32)
    npe = EPAD - E
    pad = jnp.arange(npe, dtype=jnp.int32)
    src_pad = jnp.concatenate([src, (pad * 13) % N])
    dst_pad = jnp.concatenate([dst, N + pad % (NPAD - N)])

    zeros_row = jnp.zeros((RPS, D), jnp.float32)
    zeros_deg = jnp.zeros((RPS, DEGW), jnp.float32)
    ones_deg = jnp.ones((K, DEGW), jnp.float32)

    degw = _sc_degree(dst_pad, ones_deg, zeros_deg)
    xw = _mm(x, W1)
    h1p = _scale(xw, degw)
    s1 = _sc_scatter(h1p, src_pad, dst_pad, zeros_row)
    h2p = _mid(s1, h1p, degw, b1.reshape(1, D), W2)
    s2 = _sc_scatter(h2p, src_pad, dst_pad, zeros_row)
    return _final(s2, h2p, degw, b2.reshape(1, D),
                  batch.reshape(1, N).astype(jnp.int32), ogt.reshape(G, 1),
                  Wo1, bo1.reshape(1, 20), Wo2, bo2.reshape(1, 10),
                  Wf1, bf1.reshape(1, 64), Wf2, bf2.reshape(1, 1))
